# Initial kernel scaffold; baseline (speedup 1.0000x reference)
import jax
import jax.numpy as jnp
from jax.experimental import pallas as pl


def kernel(stu_dis, tea_dis):
    raise NotImplementedError("write your pallas kernel here")


# speedup vs baseline: 12.7243x; 12.7243x over previous
import functools

import numpy as np
import jax
import jax.numpy as jnp
from jax import lax
from jax.experimental import pallas as pl
from jax.experimental.pallas import tpu as pltpu
from jax.experimental.pallas import tpu_sc as plsc

_B, _N = 128, 32768
_K = 1024
_M = 1024
_G = 2048
_R = 8
_NW = 32
_RPW = _B // _NW

_PERM = None


def _perm_table() -> np.ndarray:
    global _PERM
    if _PERM is None:
        scores = np.asarray(
            jax.random.uniform(jax.random.key(1234), (_B, _N - 1), dtype=jnp.float32)
        )
        order = np.argsort(-scores, axis=1, kind="stable")[:, :_G]
        _PERM = (order + 1).astype(np.int32)
    return _PERM


_perm_table()


def _sc_gather_body(tea_hbm, stu_hbm, perm_hbm, tea_out, stu_out,
                    row_t, row_s, idx_v, gat_t, gat_s):
    c = lax.axis_index("c")
    s = lax.axis_index("s")
    wid = s * 2 + c
    for r in range(_RPW):
        row = wid * _RPW + r
        pltpu.sync_copy(perm_hbm.at[row], idx_v)
        pltpu.sync_copy(tea_hbm.at[row], row_t)
        pltpu.sync_copy(stu_hbm.at[row], row_s)

        def body(j, carry):
            sl = pl.ds(j * 16, 16)
            idx = idx_v[sl]
            gat_t[sl] = plsc.load_gather(row_t, [idx])
            gat_s[sl] = plsc.load_gather(row_s, [idx])
            return carry

        lax.fori_loop(0, _G // 16, body, 0)
        pltpu.sync_copy(gat_t, tea_out.at[row])
        pltpu.sync_copy(gat_s, stu_out.at[row])


_SC_GATHER = None


def _sc_gather():
    global _SC_GATHER
    if _SC_GATHER is None:
        _SC_GATHER = pl.kernel(
            _sc_gather_body,
            mesh=plsc.VectorSubcoreMesh(core_axis_name="c", subcore_axis_name="s"),
            compiler_params=pltpu.CompilerParams(needs_layout_passes=False),
            out_type=[
                jax.ShapeDtypeStruct((_B, _G), jnp.float32),
                jax.ShapeDtypeStruct((_B, _G), jnp.float32),
            ],
            scratch_types=[
                pltpu.VMEM((_N,), jnp.float32),
                pltpu.VMEM((_N,), jnp.float32),
                pltpu.VMEM((_G,), jnp.int32),
                pltpu.VMEM((_G,), jnp.float32),
                pltpu.VMEM((_G,), jnp.float32),
            ],
        )
    return _SC_GATHER


def _mono_key(x):
    bu = lax.bitcast_convert_type(x, jnp.uint32)
    neg = bu >= jnp.uint32(0x80000000)
    key = jnp.where(neg, ~bu, bu | jnp.uint32(0x80000000))
    return jnp.where(x == 0.0, jnp.uint32(0x80000000), key)


def _tc_body(tea_ref, stu_ref, teag_ref, stug_ref, perm_ref, out_ref):
    pid = pl.program_id(0)
    tea = tea_ref[...]
    stu = stu_ref[...]
    col = lax.broadcasted_iota(jnp.int32, (_R, _N), 1)
    key = _mono_key(tea)
    key = jnp.where(col == 0, jnp.uint32(0), key)

    def fcount(pred):
        return jnp.sum(jnp.where(pred, 1.0, 0.0), axis=1, keepdims=True)

    def s1_body(i, carry):
        lo, hi = carry
        mid = lo + ((hi - lo + jnp.uint32(1)) >> 1)
        ge = fcount(key >= mid) >= _K
        return jnp.where(ge, mid, lo), jnp.where(ge, hi, mid - jnp.uint32(1))

    lo0 = jnp.zeros((_R, 1), jnp.uint32)
    hi0 = jnp.full((_R, 1), 0xFF800000, jnp.uint32)
    t, _ = lax.fori_loop(0, 32, s1_body, (lo0, hi0))

    g = fcount(key > t)
    need = jnp.float32(_K) - g
    eq = key == t

    def s2_body(i, carry):
        lo, hi = carry
        mid = (lo + hi) >> 1
        ge = fcount(eq & (col <= mid)) >= need
        return jnp.where(ge, lo, mid + 1), jnp.where(ge, mid, hi)

    lo1 = jnp.ones((_R, 1), jnp.int32)
    hi1 = jnp.full((_R, 1), _N - 1, jnp.int32)
    cstar, _ = lax.fori_loop(0, 15, s2_body, (lo1, hi1))

    mask_top = (key > t) | (eq & (col <= cstar))

    teag = teag_ref[...]
    stug = stug_ref[...]
    perm = perm_ref[...]
    keyg = _mono_key(teag)
    unm = ~((keyg > t) | ((keyg == t) & (perm <= cstar)))
    posg = lax.broadcasted_iota(jnp.int32, (_R, _G), 1)

    def gcount(pred):
        return jnp.sum(jnp.where(pred, 1.0, 0.0), axis=1, keepdims=True)

    def s3_body(i, carry):
        lo, hi = carry
        mid = (lo + hi) >> 1
        ge = gcount(unm & (posg < mid)) >= _M
        return jnp.where(ge, lo, mid + 1), jnp.where(ge, mid, hi)

    lo2 = jnp.full((_R, 1), _M, jnp.int32)
    hi2 = jnp.full((_R, 1), _G, jnp.int32)
    pcut, _ = lax.fori_loop(0, 11, s3_body, (lo2, hi2))
    sel = unm & (posg < pcut)

    mask_d = mask_top | (col == 0)
    ninf = jnp.float32(-jnp.inf)
    tea_m = jnp.where(mask_d, tea, ninf)
    stu_m = jnp.where(mask_d, stu, ninf)
    teag_m = jnp.where(sel, teag, ninf)
    stug_m = jnp.where(sel, stug, ninf)

    mt = jnp.maximum(jnp.max(tea_m, axis=1, keepdims=True),
                     jnp.max(teag_m, axis=1, keepdims=True))
    ms = jnp.maximum(jnp.max(stu_m, axis=1, keepdims=True),
                     jnp.max(stug_m, axis=1, keepdims=True))

    et_d = jnp.exp(tea_m - mt)
    et_g = jnp.exp(teag_m - mt)
    e_t = (jnp.sum(et_d, axis=1, keepdims=True)
           + jnp.sum(et_g, axis=1, keepdims=True))
    e_s = (jnp.sum(jnp.exp(stu_m - ms), axis=1, keepdims=True)
           + jnp.sum(jnp.exp(stug_m - ms), axis=1, keepdims=True))
    tsum = (jnp.sum(et_d * (tea - stu), axis=1, keepdims=True)
            + jnp.sum(et_g * (teag - stug), axis=1, keepdims=True))

    kl = tsum / e_t + (ms + jnp.log(e_s)) - (mt + jnp.log(e_t))
    part = (jnp.sum(kl) * jnp.float32(1.0 / _B)).reshape(1, 1)

    @pl.when(pid == 0)
    def _():
        out_ref[...] = jnp.zeros((1, 1), jnp.float32)

    out_ref[...] += part


def _tc_loss(tea, stu, tea_g, stu_g, perm):
    return pl.pallas_call(
        _tc_body,
        grid=(_B // _R,),
        in_specs=[
            pl.BlockSpec((_R, _N), lambda i: (i, 0)),
            pl.BlockSpec((_R, _N), lambda i: (i, 0)),
            pl.BlockSpec((_R, _G), lambda i: (i, 0)),
            pl.BlockSpec((_R, _G), lambda i: (i, 0)),
            pl.BlockSpec((_R, _G), lambda i: (i, 0)),
        ],
        out_specs=pl.BlockSpec((1, 1), lambda i: (0, 0)),
        out_shape=jax.ShapeDtypeStruct((1, 1), jnp.float32),
    )(tea, stu, tea_g, stu_g, perm)


def kernel(stu_dis, tea_dis):
    perm = jnp.asarray(_perm_table())
    tea_g, stu_g = _sc_gather()(tea_dis, stu_dis, perm)
    out = _tc_loss(tea_dis, stu_dis, tea_g, stu_g, perm)
    return out[0, 0]


# speedup vs baseline: 24.9149x; 1.9581x over previous
import functools

import numpy as np
import jax
import jax.numpy as jnp
from jax import lax
from jax.experimental import pallas as pl
from jax.experimental.pallas import tpu as pltpu
from jax.experimental.pallas import tpu_sc as plsc

_B, _N = 128, 32768
_K = 1024
_M = 1024
_G = 2048
_R = 16
_NW = 32
_RPW = _B // _NW

_PERM = None


def _np_threefry_uniform(seed: int, shape) -> np.ndarray:
    def rotl(x, d):
        return ((x << np.uint32(d)) | (x >> np.uint32(32 - d))).astype(np.uint32)

    size = int(np.prod(shape))
    x0 = np.zeros(size, dtype=np.uint32)
    x1 = np.arange(size, dtype=np.uint32)
    ks0 = np.uint32(seed >> 32)
    ks1 = np.uint32(seed & 0xFFFFFFFF)
    ks2 = np.uint32(np.uint32(0x1BD11BDA) ^ ks0 ^ ks1)
    x0 = (x0 + ks0).astype(np.uint32)
    x1 = (x1 + ks1).astype(np.uint32)
    rotations = [(13, 15, 26, 6), (17, 29, 16, 24)]
    keys = [(ks1, ks2), (ks2, ks0), (ks0, ks1), (ks1, ks2), (ks2, ks0)]
    for i in range(5):
        for d in rotations[i % 2]:
            x0 = (x0 + x1).astype(np.uint32)
            x1 = rotl(x1, d)
            x1 = (x1 ^ x0).astype(np.uint32)
        x0 = (x0 + keys[i][0]).astype(np.uint32)
        x1 = (x1 + keys[i][1] + np.uint32(i + 1)).astype(np.uint32)
    bits = (x0 ^ x1).reshape(shape)
    fb = ((bits >> np.uint32(9)) | np.uint32(0x3F800000)).astype(np.uint32)
    return fb.view(np.float32) - np.float32(1.0)


def _perm_table() -> np.ndarray:
    global _PERM
    if _PERM is None:
        scores = _np_threefry_uniform(1234, (_B, _N - 1))
        order = np.argsort(-scores, axis=1, kind="stable")[:, :_G]
        _PERM = (order + 1).astype(np.int32)
    return _PERM


def _sc_gather_body(tea_hbm, stu_hbm, perm_hbm, tea_out, stu_out,
                    row_t, row_s, idx_v, gat_t, gat_s):
    c = lax.axis_index("c")
    s = lax.axis_index("s")
    wid = s * 2 + c
    for r in range(_RPW):
        row = wid * _RPW + r
        pltpu.sync_copy(perm_hbm.at[row], idx_v)
        pltpu.sync_copy(tea_hbm.at[row], row_t)
        pltpu.sync_copy(stu_hbm.at[row], row_s)

        def body(j, carry):
            sl = pl.ds(j * 16, 16)
            idx = idx_v[sl]
            gat_t[sl] = plsc.load_gather(row_t, [idx])
            gat_s[sl] = plsc.load_gather(row_s, [idx])
            return carry

        lax.fori_loop(0, _G // 16, body, 0)
        pltpu.sync_copy(gat_t, tea_out.at[row])
        pltpu.sync_copy(gat_s, stu_out.at[row])


_SC_GATHER = None


def _sc_gather():
    global _SC_GATHER
    if _SC_GATHER is None:
        _SC_GATHER = pl.kernel(
            _sc_gather_body,
            mesh=plsc.VectorSubcoreMesh(core_axis_name="c", subcore_axis_name="s"),
            compiler_params=pltpu.CompilerParams(needs_layout_passes=False),
            out_type=[
                jax.ShapeDtypeStruct((_B, _G), jnp.float32),
                jax.ShapeDtypeStruct((_B, _G), jnp.float32),
            ],
            scratch_types=[
                pltpu.VMEM((_N,), jnp.float32),
                pltpu.VMEM((_N,), jnp.float32),
                pltpu.VMEM((_G,), jnp.int32),
                pltpu.VMEM((_G,), jnp.float32),
                pltpu.VMEM((_G,), jnp.float32),
            ],
        )
    return _SC_GATHER


def _mono_key(x):
    bu = lax.bitcast_convert_type(x, jnp.uint32)
    neg = bu >= jnp.uint32(0x80000000)
    key = jnp.where(neg, ~bu, bu | jnp.uint32(0x80000000))
    return jnp.where(x == 0.0, jnp.uint32(0x80000000), key)


def _tc_body(tea_ref, stu_ref, teag_ref, stug_ref, perm_ref, out_ref):
    pid = pl.program_id(0)
    tea = tea_ref[...]
    stu = stu_ref[...]
    col = lax.broadcasted_iota(jnp.int32, (_R, _N), 1)
    key = _mono_key(tea)
    key = jnp.where(col == 0, jnp.uint32(0), key)

    def fcount(pred):
        return jnp.sum(jnp.where(pred, 1.0, 0.0), axis=1, keepdims=True)

    def s1_body(i, carry):
        lo, hi = carry
        mid = lo + ((hi - lo + jnp.uint32(1)) >> 1)
        ge = fcount(key >= mid) >= _K
        return jnp.where(ge, mid, lo), jnp.where(ge, hi, mid - jnp.uint32(1))

    lo0 = jnp.zeros((_R, 1), jnp.uint32)
    hi0 = jnp.full((_R, 1), 0xFF800000, jnp.uint32)
    t, _ = lax.fori_loop(0, 32, s1_body, (lo0, hi0))

    g = fcount(key > t)
    need = jnp.float32(_K) - g
    eq = key == t
    e = fcount(eq)

    def s2_search(_):
        def s2_body(i, carry):
            lo, hi = carry
            mid = (lo + hi) >> 1
            ge = fcount(eq & (col <= mid)) >= need
            return jnp.where(ge, lo, mid + 1), jnp.where(ge, mid, hi)

        lo1 = jnp.ones((_R, 1), jnp.int32)
        hi1 = jnp.full((_R, 1), _N - 1, jnp.int32)
        cs, _ = lax.fori_loop(0, 15, s2_body, (lo1, hi1))
        return cs

    cstar = lax.cond(
        jnp.any(e != need),
        s2_search,
        lambda _: jnp.full((_R, 1), _N - 1, jnp.int32),
        operand=None,
    )

    mask_top = (key > t) | (eq & (col <= cstar))

    teag = teag_ref[...]
    stug = stug_ref[...]
    perm = perm_ref[...]
    keyg = _mono_key(teag)
    unm = ~((keyg > t) | ((keyg == t) & (perm <= cstar)))
    posg = lax.broadcasted_iota(jnp.int32, (_R, _G), 1)

    def gcount(pred):
        return jnp.sum(jnp.where(pred, 1.0, 0.0), axis=1, keepdims=True)

    def s3_body(i, carry):
        lo, hi = carry
        mid = (lo + hi) >> 1
        ge = gcount(unm & (posg < mid)) >= _M
        return jnp.where(ge, lo, mid + 1), jnp.where(ge, mid, hi)

    lo2 = jnp.full((_R, 1), _M, jnp.int32)
    hi2 = jnp.full((_R, 1), _G, jnp.int32)
    pcut, _ = lax.fori_loop(0, 11, s3_body, (lo2, hi2))
    sel = unm & (posg < pcut)

    mask_d = mask_top | (col == 0)
    ninf = jnp.float32(-jnp.inf)
    tea_m = jnp.where(mask_d, tea, ninf)
    stu_m = jnp.where(mask_d, stu, ninf)
    teag_m = jnp.where(sel, teag, ninf)
    stug_m = jnp.where(sel, stug, ninf)

    mt = jnp.maximum(jnp.max(tea_m, axis=1, keepdims=True),
                     jnp.max(teag_m, axis=1, keepdims=True))
    ms = jnp.maximum(jnp.max(stu_m, axis=1, keepdims=True),
                     jnp.max(stug_m, axis=1, keepdims=True))

    et_d = jnp.exp(tea_m - mt)
    et_g = jnp.exp(teag_m - mt)
    e_t = (jnp.sum(et_d, axis=1, keepdims=True)
           + jnp.sum(et_g, axis=1, keepdims=True))
    e_s = (jnp.sum(jnp.exp(stu_m - ms), axis=1, keepdims=True)
           + jnp.sum(jnp.exp(stug_m - ms), axis=1, keepdims=True))
    tsum = (jnp.sum(et_d * (tea - stu), axis=1, keepdims=True)
            + jnp.sum(et_g * (teag - stug), axis=1, keepdims=True))

    kl = tsum / e_t + (ms + jnp.log(e_s)) - (mt + jnp.log(e_t))
    part = (jnp.sum(kl) * jnp.float32(1.0 / _B)).reshape(1, 1)

    @pl.when(pid == 0)
    def _():
        out_ref[...] = jnp.zeros((1, 1), jnp.float32)

    out_ref[...] += part


def _tc_loss(tea, stu, tea_g, stu_g, perm):
    return pl.pallas_call(
        _tc_body,
        grid=(_B // _R,),
        in_specs=[
            pl.BlockSpec((_R, _N), lambda i: (i, 0)),
            pl.BlockSpec((_R, _N), lambda i: (i, 0)),
            pl.BlockSpec((_R, _G), lambda i: (i, 0)),
            pl.BlockSpec((_R, _G), lambda i: (i, 0)),
            pl.BlockSpec((_R, _G), lambda i: (i, 0)),
        ],
        out_specs=pl.BlockSpec((1, 1), lambda i: (0, 0)),
        out_shape=jax.ShapeDtypeStruct((1, 1), jnp.float32),
    )(tea, stu, tea_g, stu_g, perm)


def kernel(stu_dis, tea_dis):
    perm = jnp.asarray(_perm_table())
    tea_g, stu_g = _sc_gather()(tea_dis, stu_dis, perm)
    out = _tc_loss(tea_dis, stu_dis, tea_g, stu_g, perm)
    return out[0, 0]


# speedup vs baseline: 27.7192x; 1.1126x over previous
import functools

import numpy as np
import jax
import jax.numpy as jnp
from jax import lax
from jax.experimental import pallas as pl
from jax.experimental.pallas import tpu as pltpu
from jax.experimental.pallas import tpu_sc as plsc

_B, _N = 128, 32768
_K = 1024
_M = 1024
_G = 2048
_R = 32
_NW = 32
_RPW = _B // _NW

_PERM = None


def _np_threefry_uniform(seed: int, shape) -> np.ndarray:
    def rotl(x, d):
        return ((x << np.uint32(d)) | (x >> np.uint32(32 - d))).astype(np.uint32)

    size = int(np.prod(shape))
    x0 = np.zeros(size, dtype=np.uint32)
    x1 = np.arange(size, dtype=np.uint32)
    ks0 = np.uint32(seed >> 32)
    ks1 = np.uint32(seed & 0xFFFFFFFF)
    ks2 = np.uint32(np.uint32(0x1BD11BDA) ^ ks0 ^ ks1)
    x0 = (x0 + ks0).astype(np.uint32)
    x1 = (x1 + ks1).astype(np.uint32)
    rotations = [(13, 15, 26, 6), (17, 29, 16, 24)]
    keys = [(ks1, ks2), (ks2, ks0), (ks0, ks1), (ks1, ks2), (ks2, ks0)]
    for i in range(5):
        for d in rotations[i % 2]:
            x0 = (x0 + x1).astype(np.uint32)
            x1 = rotl(x1, d)
            x1 = (x1 ^ x0).astype(np.uint32)
        x0 = (x0 + keys[i][0]).astype(np.uint32)
        x1 = (x1 + keys[i][1] + np.uint32(i + 1)).astype(np.uint32)
    bits = (x0 ^ x1).reshape(shape)
    fb = ((bits >> np.uint32(9)) | np.uint32(0x3F800000)).astype(np.uint32)
    return fb.view(np.float32) - np.float32(1.0)


def _perm_table() -> np.ndarray:
    global _PERM
    if _PERM is None:
        scores = _np_threefry_uniform(1234, (_B, _N - 1))
        order = np.argsort(-scores, axis=1, kind="stable")[:, :_G]
        _PERM = (order + 1).astype(np.int32)
    return _PERM


def _sc_gather_body(tea_hbm, stu_hbm, perm_hbm, tea_out, stu_out,
                    row_t, row_s, idx_v, gat_t, gat_s):
    c = lax.axis_index("c")
    s = lax.axis_index("s")
    wid = s * 2 + c
    for r in range(_RPW):
        row = wid * _RPW + r
        pltpu.sync_copy(perm_hbm.at[row], idx_v)
        pltpu.sync_copy(tea_hbm.at[row], row_t)
        pltpu.sync_copy(stu_hbm.at[row], row_s)

        def body(j, carry):
            sl = pl.ds(j * 16, 16)
            idx = idx_v[sl]
            gat_t[sl] = plsc.load_gather(row_t, [idx])
            gat_s[sl] = plsc.load_gather(row_s, [idx])
            return carry

        lax.fori_loop(0, _G // 16, body, 0)
        pltpu.sync_copy(gat_t, tea_out.at[row])
        pltpu.sync_copy(gat_s, stu_out.at[row])


_SC_GATHER = None


def _sc_gather():
    global _SC_GATHER
    if _SC_GATHER is None:
        _SC_GATHER = pl.kernel(
            _sc_gather_body,
            mesh=plsc.VectorSubcoreMesh(core_axis_name="c", subcore_axis_name="s"),
            compiler_params=pltpu.CompilerParams(needs_layout_passes=False),
            out_type=[
                jax.ShapeDtypeStruct((_B, _G), jnp.float32),
                jax.ShapeDtypeStruct((_B, _G), jnp.float32),
            ],
            scratch_types=[
                pltpu.VMEM((_N,), jnp.float32),
                pltpu.VMEM((_N,), jnp.float32),
                pltpu.VMEM((_G,), jnp.int32),
                pltpu.VMEM((_G,), jnp.float32),
                pltpu.VMEM((_G,), jnp.float32),
            ],
        )
    return _SC_GATHER


def _mono_key(x):
    bu = lax.bitcast_convert_type(x, jnp.uint32)
    neg = bu >= jnp.uint32(0x80000000)
    key = jnp.where(neg, ~bu, bu | jnp.uint32(0x80000000))
    return jnp.where(x == 0.0, jnp.uint32(0x80000000), key)


def _tc_body(tea_ref, stu_ref, teag_ref, stug_ref, perm_ref, out_ref):
    pid = pl.program_id(0)
    tea = tea_ref[...]
    stu = stu_ref[...]
    col = lax.broadcasted_iota(jnp.int32, (_R, _N), 1)
    key = _mono_key(tea)
    key = jnp.where(col == 0, jnp.uint32(0), key)

    def fcount(pred):
        return jnp.sum(jnp.where(pred, 1.0, 0.0), axis=1, keepdims=True)

    def s1_body(i, carry):
        lo, hi = carry
        mid = lo + ((hi - lo + jnp.uint32(1)) >> 1)
        ge = fcount(key >= mid) >= _K
        return jnp.where(ge, mid, lo), jnp.where(ge, hi, mid - jnp.uint32(1))

    lo0 = jnp.zeros((_R, 1), jnp.uint32)
    hi0 = jnp.full((_R, 1), 0xFF800000, jnp.uint32)
    t, _ = lax.fori_loop(0, 32, s1_body, (lo0, hi0))

    g = fcount(key > t)
    need = jnp.float32(_K) - g
    eq = key == t
    e = fcount(eq)

    def s2_search(_):
        def s2_body(i, carry):
            lo, hi = carry
            mid = (lo + hi) >> 1
            ge = fcount(eq & (col <= mid)) >= need
            return jnp.where(ge, lo, mid + 1), jnp.where(ge, mid, hi)

        lo1 = jnp.ones((_R, 1), jnp.int32)
        hi1 = jnp.full((_R, 1), _N - 1, jnp.int32)
        cs, _ = lax.fori_loop(0, 15, s2_body, (lo1, hi1))
        return cs

    cstar = lax.cond(
        jnp.any(e != need),
        s2_search,
        lambda _: jnp.full((_R, 1), _N - 1, jnp.int32),
        operand=None,
    )

    mask_top = (key > t) | (eq & (col <= cstar))

    teag = teag_ref[...]
    stug = stug_ref[...]
    perm = perm_ref[...]
    keyg = _mono_key(teag)
    unm = ~((keyg > t) | ((keyg == t) & (perm <= cstar)))
    posg = lax.broadcasted_iota(jnp.int32, (_R, _G), 1)

    def gcount(pred):
        return jnp.sum(jnp.where(pred, 1.0, 0.0), axis=1, keepdims=True)

    def s3_body(i, carry):
        lo, hi = carry
        mid = (lo + hi) >> 1
        ge = gcount(unm & (posg < mid)) >= _M
        return jnp.where(ge, lo, mid + 1), jnp.where(ge, mid, hi)

    lo2 = jnp.full((_R, 1), _M, jnp.int32)
    hi2 = jnp.full((_R, 1), _G, jnp.int32)
    pcut, _ = lax.fori_loop(0, 11, s3_body, (lo2, hi2))
    sel = unm & (posg < pcut)

    mask_d = mask_top | (col == 0)
    ninf = jnp.float32(-jnp.inf)
    tea_m = jnp.where(mask_d, tea, ninf)
    stu_m = jnp.where(mask_d, stu, ninf)
    teag_m = jnp.where(sel, teag, ninf)
    stug_m = jnp.where(sel, stug, ninf)

    mt = jnp.maximum(jnp.max(tea_m, axis=1, keepdims=True),
                     jnp.max(teag_m, axis=1, keepdims=True))
    ms = jnp.maximum(jnp.max(stu_m, axis=1, keepdims=True),
                     jnp.max(stug_m, axis=1, keepdims=True))

    et_d = jnp.exp(tea_m - mt)
    et_g = jnp.exp(teag_m - mt)
    e_t = (jnp.sum(et_d, axis=1, keepdims=True)
           + jnp.sum(et_g, axis=1, keepdims=True))
    e_s = (jnp.sum(jnp.exp(stu_m - ms), axis=1, keepdims=True)
           + jnp.sum(jnp.exp(stug_m - ms), axis=1, keepdims=True))
    tsum = (jnp.sum(et_d * (tea - stu), axis=1, keepdims=True)
            + jnp.sum(et_g * (teag - stug), axis=1, keepdims=True))

    kl = tsum / e_t + (ms + jnp.log(e_s)) - (mt + jnp.log(e_t))
    part = (jnp.sum(kl) * jnp.float32(1.0 / _B)).reshape(1, 1)

    @pl.when(pid == 0)
    def _():
        out_ref[...] = jnp.zeros((1, 1), jnp.float32)

    out_ref[...] += part


def _tc_loss(tea, stu, tea_g, stu_g, perm):
    return pl.pallas_call(
        _tc_body,
        grid=(_B // _R,),
        in_specs=[
            pl.BlockSpec((_R, _N), lambda i: (i, 0)),
            pl.BlockSpec((_R, _N), lambda i: (i, 0)),
            pl.BlockSpec((_R, _G), lambda i: (i, 0)),
            pl.BlockSpec((_R, _G), lambda i: (i, 0)),
            pl.BlockSpec((_R, _G), lambda i: (i, 0)),
        ],
        out_specs=pl.BlockSpec((1, 1), lambda i: (0, 0)),
        out_shape=jax.ShapeDtypeStruct((1, 1), jnp.float32),
    )(tea, stu, tea_g, stu_g, perm)


def kernel(stu_dis, tea_dis):
    perm = jnp.asarray(_perm_table())
    tea_g, stu_g = _sc_gather()(tea_dis, stu_dis, perm)
    out = _tc_loss(tea_dis, stu_dis, tea_g, stu_g, perm)
    return out[0, 0]


# speedup vs baseline: 33.2560x; 1.1997x over previous
import functools

import numpy as np
import jax
import jax.numpy as jnp
from jax import lax
from jax.experimental import pallas as pl
from jax.experimental.pallas import tpu as pltpu
from jax.experimental.pallas import tpu_sc as plsc

_B, _N = 128, 32768
_K = 1024
_M = 1024
_G = 2048
_R = 32
_NW = 32
_RPW = _B // _NW

_PERM = None


def _np_threefry_uniform(seed: int, shape) -> np.ndarray:
    def rotl(x, d):
        return ((x << np.uint32(d)) | (x >> np.uint32(32 - d))).astype(np.uint32)

    size = int(np.prod(shape))
    x0 = np.zeros(size, dtype=np.uint32)
    x1 = np.arange(size, dtype=np.uint32)
    ks0 = np.uint32(seed >> 32)
    ks1 = np.uint32(seed & 0xFFFFFFFF)
    ks2 = np.uint32(np.uint32(0x1BD11BDA) ^ ks0 ^ ks1)
    x0 = (x0 + ks0).astype(np.uint32)
    x1 = (x1 + ks1).astype(np.uint32)
    rotations = [(13, 15, 26, 6), (17, 29, 16, 24)]
    keys = [(ks1, ks2), (ks2, ks0), (ks0, ks1), (ks1, ks2), (ks2, ks0)]
    for i in range(5):
        for d in rotations[i % 2]:
            x0 = (x0 + x1).astype(np.uint32)
            x1 = rotl(x1, d)
            x1 = (x1 ^ x0).astype(np.uint32)
        x0 = (x0 + keys[i][0]).astype(np.uint32)
        x1 = (x1 + keys[i][1] + np.uint32(i + 1)).astype(np.uint32)
    bits = (x0 ^ x1).reshape(shape)
    fb = ((bits >> np.uint32(9)) | np.uint32(0x3F800000)).astype(np.uint32)
    return fb.view(np.float32) - np.float32(1.0)


def _perm_table() -> np.ndarray:
    global _PERM
    if _PERM is None:
        scores = _np_threefry_uniform(1234, (_B, _N - 1))
        order = np.argsort(-scores, axis=1, kind="stable")[:, :_G]
        _PERM = (order + 1).astype(np.int32)
    return _PERM


def _sc_gather_body(tea_hbm, stu_hbm, perm_hbm, tea_out, stu_out,
                    row_t, row_s, idx_v, gat_t, gat_s):
    c = lax.axis_index("c")
    s = lax.axis_index("s")
    wid = s * 2 + c
    for r in range(_RPW):
        row = wid * _RPW + r
        pltpu.sync_copy(perm_hbm.at[row], idx_v)
        pltpu.sync_copy(tea_hbm.at[row], row_t)
        pltpu.sync_copy(stu_hbm.at[row], row_s)

        def body(j, carry):
            sl = pl.ds(j * 16, 16)
            idx = idx_v[sl]
            gat_t[sl] = plsc.load_gather(row_t, [idx])
            gat_s[sl] = plsc.load_gather(row_s, [idx])
            return carry

        lax.fori_loop(0, _G // 16, body, 0)
        pltpu.sync_copy(gat_t, tea_out.at[row])
        pltpu.sync_copy(gat_s, stu_out.at[row])


_SC_GATHER = None


def _sc_gather():
    global _SC_GATHER
    if _SC_GATHER is None:
        _SC_GATHER = pl.kernel(
            _sc_gather_body,
            mesh=plsc.VectorSubcoreMesh(core_axis_name="c", subcore_axis_name="s"),
            compiler_params=pltpu.CompilerParams(needs_layout_passes=False),
            out_type=[
                jax.ShapeDtypeStruct((_B, _G), jnp.float32),
                jax.ShapeDtypeStruct((_B, _G), jnp.float32),
            ],
            scratch_types=[
                pltpu.VMEM((_N,), jnp.float32),
                pltpu.VMEM((_N,), jnp.float32),
                pltpu.VMEM((_G,), jnp.int32),
                pltpu.VMEM((_G,), jnp.float32),
                pltpu.VMEM((_G,), jnp.float32),
            ],
        )
    return _SC_GATHER


def _mono_key(x):
    bu = lax.bitcast_convert_type(x, jnp.uint32)
    neg = bu >= jnp.uint32(0x80000000)
    key = jnp.where(neg, ~bu, bu | jnp.uint32(0x80000000))
    return jnp.where(x == 0.0, jnp.uint32(0x80000000), key)


_CW1 = 512
_CW2 = 256


def _fold128(v, op):
    r = v[:, 0:128]
    for off in range(128, v.shape[1], 128):
        r = op(r, v[:, off:off + 128])
    return r


def _tc_body(tea_ref, stu_ref, teag_ref, stug_ref, perm_ref, out_ref, key_ref):
    pid = pl.program_id(0)
    nch1 = _N // _CW1
    iota1 = lax.broadcasted_iota(jnp.int32, (_R, _CW1), 1)
    add = lambda a, b: a + b

    mt_acc = jnp.full((_R, 128), -jnp.inf, jnp.float32)
    ms_acc = jnp.full((_R, 128), -jnp.inf, jnp.float32)
    for c in range(nch1):
        sl = pl.ds(c * _CW1, _CW1)
        x = tea_ref[:, sl]
        s = stu_ref[:, sl]
        k = _mono_key(x)
        if c == 0:
            k = jnp.where(iota1 == 0, jnp.uint32(0), k)
        key_ref[:, sl] = k
        mt_acc = jnp.maximum(mt_acc, _fold128(x, jnp.maximum))
        ms_acc = jnp.maximum(ms_acc, _fold128(s, jnp.maximum))
    mt = jnp.max(mt_acc, axis=1, keepdims=True)
    ms = jnp.max(ms_acc, axis=1, keepdims=True)

    def count(pred_fn):
        acc = jnp.zeros((_R, 128), jnp.float32)
        for c in range(nch1):
            kc = key_ref[:, pl.ds(c * _CW1, _CW1)]
            m = jnp.where(pred_fn(kc, c), 1.0, 0.0)
            acc = acc + _fold128(m, add)
        return jnp.sum(acc, axis=1, keepdims=True)

    def s1_body(i, carry):
        lo, hi = carry
        mid = lo + ((hi - lo + jnp.uint32(1)) >> 1)
        ge = count(lambda kc, c: kc >= mid) >= _K
        return jnp.where(ge, mid, lo), jnp.where(ge, hi, mid - jnp.uint32(1))

    lo0 = jnp.zeros((_R, 1), jnp.uint32)
    hi0 = jnp.full((_R, 1), 0xFF800000, jnp.uint32)
    t, _ = lax.fori_loop(0, 32, s1_body, (lo0, hi0))

    accg = jnp.zeros((_R, 128), jnp.float32)
    acce = jnp.zeros((_R, 128), jnp.float32)
    for c in range(nch1):
        kc = key_ref[:, pl.ds(c * _CW1, _CW1)]
        accg = accg + _fold128(jnp.where(kc > t, 1.0, 0.0), add)
        acce = acce + _fold128(jnp.where(kc == t, 1.0, 0.0), add)
    g = jnp.sum(accg, axis=1, keepdims=True)
    e = jnp.sum(acce, axis=1, keepdims=True)
    need = jnp.float32(_K) - g

    def s2_search(_):
        def s2_body(i, carry):
            lo, hi = carry
            mid = (lo + hi) >> 1
            ge = count(
                lambda kc, c: (kc == t) & ((iota1 + c * _CW1) <= mid)
            ) >= need
            return jnp.where(ge, lo, mid + 1), jnp.where(ge, mid, hi)

        lo1 = jnp.ones((_R, 1), jnp.int32)
        hi1 = jnp.full((_R, 1), _N - 1, jnp.int32)
        cs, _ = lax.fori_loop(0, 15, s2_body, (lo1, hi1))
        return cs

    cstar = lax.cond(
        jnp.any(e != need),
        s2_search,
        lambda _: jnp.full((_R, 1), _N - 1, jnp.int32),
        operand=None,
    )

    teag = teag_ref[...]
    stug = stug_ref[...]
    perm = perm_ref[...]
    keyg = _mono_key(teag)
    unm = ~((keyg > t) | ((keyg == t) & (perm <= cstar)))
    posg = lax.broadcasted_iota(jnp.int32, (_R, _G), 1)

    def gcount(pred):
        return jnp.sum(jnp.where(pred, 1.0, 0.0), axis=1, keepdims=True)

    def s3_body(i, carry):
        lo, hi = carry
        mid = (lo + hi) >> 1
        ge = gcount(unm & (posg < mid)) >= _M
        return jnp.where(ge, lo, mid + 1), jnp.where(ge, mid, hi)

    lo2 = jnp.full((_R, 1), _M, jnp.int32)
    hi2 = jnp.full((_R, 1), _G, jnp.int32)
    pcut, _ = lax.fori_loop(0, 11, s3_body, (lo2, hi2))
    sel = unm & (posg < pcut)

    nch2 = _N // _CW2
    iota2 = lax.broadcasted_iota(jnp.int32, (_R, _CW2), 1)
    acc_et = jnp.zeros((_R, 128), jnp.float32)
    acc_es = jnp.zeros((_R, 128), jnp.float32)
    acc_ts = jnp.zeros((_R, 128), jnp.float32)
    for c in range(nch2):
        sl = pl.ds(c * _CW2, _CW2)
        kc = key_ref[:, sl]
        x = tea_ref[:, sl]
        s = stu_ref[:, sl]
        m = (kc > t) | ((kc == t) & ((iota2 + c * _CW2) <= cstar))
        if c == 0:
            m = m | (iota2 == 0)
        w = jnp.where(m, jnp.exp(x - mt), 0.0)
        acc_et = acc_et + _fold128(w, add)
        acc_ts = acc_ts + _fold128(w * (x - s), add)
        acc_es = acc_es + _fold128(jnp.where(m, jnp.exp(s - ms), 0.0), add)
    e_t = jnp.sum(acc_et, axis=1, keepdims=True)
    e_s = jnp.sum(acc_es, axis=1, keepdims=True)
    tsum = jnp.sum(acc_ts, axis=1, keepdims=True)

    wg = jnp.where(sel, jnp.exp(teag - mt), 0.0)
    e_t = e_t + jnp.sum(wg, axis=1, keepdims=True)
    tsum = tsum + jnp.sum(wg * (teag - stug), axis=1, keepdims=True)
    e_s = e_s + jnp.sum(jnp.where(sel, jnp.exp(stug - ms), 0.0),
                        axis=1, keepdims=True)

    kl = tsum / e_t + (ms + jnp.log(e_s)) - (mt + jnp.log(e_t))
    part = (jnp.sum(kl) * jnp.float32(1.0 / _B)).reshape(1, 1)

    @pl.when(pid == 0)
    def _():
        out_ref[...] = jnp.zeros((1, 1), jnp.float32)

    out_ref[...] += part


def _tc_loss(tea, stu, tea_g, stu_g, perm):
    return pl.pallas_call(
        _tc_body,
        grid=(_B // _R,),
        in_specs=[
            pl.BlockSpec((_R, _N), lambda i: (i, 0)),
            pl.BlockSpec((_R, _N), lambda i: (i, 0)),
            pl.BlockSpec((_R, _G), lambda i: (i, 0)),
            pl.BlockSpec((_R, _G), lambda i: (i, 0)),
            pl.BlockSpec((_R, _G), lambda i: (i, 0)),
        ],
        out_specs=pl.BlockSpec((1, 1), lambda i: (0, 0)),
        out_shape=jax.ShapeDtypeStruct((1, 1), jnp.float32),
        scratch_shapes=[pltpu.VMEM((_R, _N), jnp.uint32)],
    )(tea, stu, tea_g, stu_g, perm)


def kernel(stu_dis, tea_dis):
    perm = jnp.asarray(_perm_table())
    tea_g, stu_g = _sc_gather()(tea_dis, stu_dis, perm)
    out = _tc_loss(tea_dis, stu_dis, tea_g, stu_g, perm)
    return out[0, 0]


# speedup vs baseline: 38.5854x; 1.1603x over previous
import functools

import numpy as np
import jax
import jax.numpy as jnp
from jax import lax
from jax.experimental import pallas as pl
from jax.experimental.pallas import tpu as pltpu
from jax.experimental.pallas import tpu_sc as plsc

_B, _N = 128, 32768
_K = 1024
_M = 1024
_G = 2048
_R = 32
_NW = 32
_RPW = _B // _NW

_PERM = None


def _np_threefry_uniform(seed: int, shape) -> np.ndarray:
    def rotl(x, d):
        return ((x << np.uint32(d)) | (x >> np.uint32(32 - d))).astype(np.uint32)

    size = int(np.prod(shape))
    x0 = np.zeros(size, dtype=np.uint32)
    x1 = np.arange(size, dtype=np.uint32)
    ks0 = np.uint32(seed >> 32)
    ks1 = np.uint32(seed & 0xFFFFFFFF)
    ks2 = np.uint32(np.uint32(0x1BD11BDA) ^ ks0 ^ ks1)
    x0 = (x0 + ks0).astype(np.uint32)
    x1 = (x1 + ks1).astype(np.uint32)
    rotations = [(13, 15, 26, 6), (17, 29, 16, 24)]
    keys = [(ks1, ks2), (ks2, ks0), (ks0, ks1), (ks1, ks2), (ks2, ks0)]
    for i in range(5):
        for d in rotations[i % 2]:
            x0 = (x0 + x1).astype(np.uint32)
            x1 = rotl(x1, d)
            x1 = (x1 ^ x0).astype(np.uint32)
        x0 = (x0 + keys[i][0]).astype(np.uint32)
        x1 = (x1 + keys[i][1] + np.uint32(i + 1)).astype(np.uint32)
    bits = (x0 ^ x1).reshape(shape)
    fb = ((bits >> np.uint32(9)) | np.uint32(0x3F800000)).astype(np.uint32)
    return fb.view(np.float32) - np.float32(1.0)


def _perm_table() -> np.ndarray:
    global _PERM
    if _PERM is None:
        scores = _np_threefry_uniform(1234, (_B, _N - 1))
        order = np.argsort(-scores, axis=1, kind="stable")[:, :_G]
        _PERM = (order + 1).astype(np.int32)
    return _PERM


def _sc_gather_body(tea_hbm, stu_hbm, perm_hbm, tea_out, stu_out,
                    row_t, row_s, idx_v, gat_t, gat_s):
    c = lax.axis_index("c")
    s = lax.axis_index("s")
    wid = s * 2 + c
    for r in range(_RPW):
        row = wid * _RPW + r
        pltpu.sync_copy(perm_hbm.at[row], idx_v)
        pltpu.sync_copy(tea_hbm.at[row], row_t)
        pltpu.sync_copy(stu_hbm.at[row], row_s)

        def body(j, carry):
            sl = pl.ds(j * 16, 16)
            idx = idx_v[sl]
            gat_t[sl] = plsc.load_gather(row_t, [idx])
            gat_s[sl] = plsc.load_gather(row_s, [idx])
            return carry

        lax.fori_loop(0, _G // 16, body, 0)
        pltpu.sync_copy(gat_t, tea_out.at[row])
        pltpu.sync_copy(gat_s, stu_out.at[row])


_SC_GATHER = None


def _sc_gather():
    global _SC_GATHER
    if _SC_GATHER is None:
        _SC_GATHER = pl.kernel(
            _sc_gather_body,
            mesh=plsc.VectorSubcoreMesh(core_axis_name="c", subcore_axis_name="s"),
            compiler_params=pltpu.CompilerParams(needs_layout_passes=False),
            out_type=[
                jax.ShapeDtypeStruct((_B, _G), jnp.float32),
                jax.ShapeDtypeStruct((_B, _G), jnp.float32),
            ],
            scratch_types=[
                pltpu.VMEM((_N,), jnp.float32),
                pltpu.VMEM((_N,), jnp.float32),
                pltpu.VMEM((_G,), jnp.int32),
                pltpu.VMEM((_G,), jnp.float32),
                pltpu.VMEM((_G,), jnp.float32),
            ],
        )
    return _SC_GATHER


def _mono_key(x):
    bu = lax.bitcast_convert_type(x, jnp.uint32)
    neg = bu >= jnp.uint32(0x80000000)
    key = jnp.where(neg, ~bu, bu | jnp.uint32(0x80000000))
    return jnp.where(x == 0.0, jnp.uint32(0x80000000), key)


_CW1 = 512
_CW2 = 256


def _fold128(v, op):
    r = v[:, 0:128]
    for off in range(128, v.shape[1], 128):
        r = op(r, v[:, off:off + 128])
    return r


def _tc_select_body(tea_ref, stu_ref, t_ref, cstar_ref, mt_ref, ms_ref,
                    key_ref):
    nch1 = _N // _CW1
    iota1 = lax.broadcasted_iota(jnp.int32, (_R, _CW1), 1)
    add = lambda a, b: a + b

    mt_acc = jnp.full((_R, 128), -jnp.inf, jnp.float32)
    ms_acc = jnp.full((_R, 128), -jnp.inf, jnp.float32)
    for c in range(nch1):
        sl = pl.ds(c * _CW1, _CW1)
        x = tea_ref[:, sl]
        s = stu_ref[:, sl]
        k = _mono_key(x)
        if c == 0:
            k = jnp.where(iota1 == 0, jnp.uint32(0), k)
        key_ref[:, sl] = k
        mt_acc = jnp.maximum(mt_acc, _fold128(x, jnp.maximum))
        ms_acc = jnp.maximum(ms_acc, _fold128(s, jnp.maximum))
    mt = jnp.max(mt_acc, axis=1, keepdims=True)
    ms = jnp.max(ms_acc, axis=1, keepdims=True)

    def count(pred_fn):
        acc = jnp.zeros((_R, 128), jnp.float32)
        for c in range(nch1):
            kc = key_ref[:, pl.ds(c * _CW1, _CW1)]
            m = jnp.where(pred_fn(kc, c), 1.0, 0.0)
            acc = acc + _fold128(m, add)
        return jnp.sum(acc, axis=1, keepdims=True)

    def s1_body(i, carry):
        lo, hi = carry
        mid = lo + ((hi - lo + jnp.uint32(1)) >> 1)
        ge = count(lambda kc, c: kc >= mid) >= _K
        return jnp.where(ge, mid, lo), jnp.where(ge, hi, mid - jnp.uint32(1))

    lo0 = jnp.zeros((_R, 1), jnp.uint32)
    hi0 = jnp.full((_R, 1), 0xFF800000, jnp.uint32)
    t, _ = lax.fori_loop(0, 32, s1_body, (lo0, hi0))

    accg = jnp.zeros((_R, 128), jnp.float32)
    acce = jnp.zeros((_R, 128), jnp.float32)
    for c in range(nch1):
        kc = key_ref[:, pl.ds(c * _CW1, _CW1)]
        accg = accg + _fold128(jnp.where(kc > t, 1.0, 0.0), add)
        acce = acce + _fold128(jnp.where(kc == t, 1.0, 0.0), add)
    g = jnp.sum(accg, axis=1, keepdims=True)
    e = jnp.sum(acce, axis=1, keepdims=True)
    need = jnp.float32(_K) - g

    def s2_search(_):
        def s2_body(i, carry):
            lo, hi = carry
            mid = (lo + hi) >> 1
            ge = count(
                lambda kc, c: (kc == t) & ((iota1 + c * _CW1) <= mid)
            ) >= need
            return jnp.where(ge, lo, mid + 1), jnp.where(ge, mid, hi)

        lo1 = jnp.ones((_R, 1), jnp.int32)
        hi1 = jnp.full((_R, 1), _N - 1, jnp.int32)
        cs, _ = lax.fori_loop(0, 15, s2_body, (lo1, hi1))
        return cs

    cstar = lax.cond(
        jnp.any(e != need),
        s2_search,
        lambda _: jnp.full((_R, 1), _N - 1, jnp.int32),
        operand=None,
    )

    t_ref[...] = t
    cstar_ref[...] = cstar
    mt_ref[...] = mt
    ms_ref[...] = ms


def _tc_final_body(tea_ref, stu_ref, teag_ref, stug_ref, perm_ref,
                   t_ref, cstar_ref, mt_ref, ms_ref, out_ref):
    pid = pl.program_id(0)
    add = lambda a, b: a + b
    t = t_ref[...]
    cstar = cstar_ref[...]
    mt = mt_ref[...]
    ms = ms_ref[...]

    teag = teag_ref[...]
    stug = stug_ref[...]
    perm = perm_ref[...]
    keyg = _mono_key(teag)
    unm = ~((keyg > t) | ((keyg == t) & (perm <= cstar)))
    posg = lax.broadcasted_iota(jnp.int32, (_R, _G), 1)

    def gcount(pred):
        return jnp.sum(jnp.where(pred, 1.0, 0.0), axis=1, keepdims=True)

    def s3_body(i, carry):
        lo, hi = carry
        mid = (lo + hi) >> 1
        ge = gcount(unm & (posg < mid)) >= _M
        return jnp.where(ge, lo, mid + 1), jnp.where(ge, mid, hi)

    lo2 = jnp.full((_R, 1), _M, jnp.int32)
    hi2 = jnp.full((_R, 1), _G, jnp.int32)
    pcut, _ = lax.fori_loop(0, 11, s3_body, (lo2, hi2))
    sel = unm & (posg < pcut)

    nch2 = _N // _CW2
    iota2 = lax.broadcasted_iota(jnp.int32, (_R, _CW2), 1)
    acc_et = jnp.zeros((_R, 128), jnp.float32)
    acc_es = jnp.zeros((_R, 128), jnp.float32)
    acc_ts = jnp.zeros((_R, 128), jnp.float32)
    for c in range(nch2):
        sl = pl.ds(c * _CW2, _CW2)
        x = tea_ref[:, sl]
        s = stu_ref[:, sl]
        kc = _mono_key(x)
        m = (kc > t) | ((kc == t) & ((iota2 + c * _CW2) <= cstar))
        if c == 0:
            m = m | (iota2 == 0)
        w = jnp.where(m, jnp.exp(x - mt), 0.0)
        acc_et = acc_et + _fold128(w, add)
        acc_ts = acc_ts + _fold128(w * (x - s), add)
        acc_es = acc_es + _fold128(jnp.where(m, jnp.exp(s - ms), 0.0), add)
    e_t = jnp.sum(acc_et, axis=1, keepdims=True)
    e_s = jnp.sum(acc_es, axis=1, keepdims=True)
    tsum = jnp.sum(acc_ts, axis=1, keepdims=True)

    wg = jnp.where(sel, jnp.exp(teag - mt), 0.0)
    e_t = e_t + jnp.sum(wg, axis=1, keepdims=True)
    tsum = tsum + jnp.sum(wg * (teag - stug), axis=1, keepdims=True)
    e_s = e_s + jnp.sum(jnp.where(sel, jnp.exp(stug - ms), 0.0),
                        axis=1, keepdims=True)

    kl = tsum / e_t + (ms + jnp.log(e_s)) - (mt + jnp.log(e_t))
    part = (jnp.sum(kl) * jnp.float32(1.0 / _B)).reshape(1, 1)

    @pl.when(pid == 0)
    def _():
        out_ref[...] = jnp.zeros((1, 1), jnp.float32)

    out_ref[...] += part


def _tc_select(tea, stu):
    return pl.pallas_call(
        _tc_select_body,
        grid=(_B // _R,),
        in_specs=[
            pl.BlockSpec((_R, _N), lambda i: (i, 0)),
            pl.BlockSpec((_R, _N), lambda i: (i, 0)),
        ],
        out_specs=[
            pl.BlockSpec((_R, 1), lambda i: (i, 0)),
            pl.BlockSpec((_R, 1), lambda i: (i, 0)),
            pl.BlockSpec((_R, 1), lambda i: (i, 0)),
            pl.BlockSpec((_R, 1), lambda i: (i, 0)),
        ],
        out_shape=[
            jax.ShapeDtypeStruct((_B, 1), jnp.uint32),
            jax.ShapeDtypeStruct((_B, 1), jnp.int32),
            jax.ShapeDtypeStruct((_B, 1), jnp.float32),
            jax.ShapeDtypeStruct((_B, 1), jnp.float32),
        ],
        scratch_shapes=[pltpu.VMEM((_R, _N), jnp.uint32)],
    )(tea, stu)


def _tc_final(tea, stu, tea_g, stu_g, perm, t, cstar, mt, ms):
    return pl.pallas_call(
        _tc_final_body,
        grid=(_B // _R,),
        in_specs=[
            pl.BlockSpec((_R, _N), lambda i: (i, 0)),
            pl.BlockSpec((_R, _N), lambda i: (i, 0)),
            pl.BlockSpec((_R, _G), lambda i: (i, 0)),
            pl.BlockSpec((_R, _G), lambda i: (i, 0)),
            pl.BlockSpec((_R, _G), lambda i: (i, 0)),
            pl.BlockSpec((_R, 1), lambda i: (i, 0)),
            pl.BlockSpec((_R, 1), lambda i: (i, 0)),
            pl.BlockSpec((_R, 1), lambda i: (i, 0)),
            pl.BlockSpec((_R, 1), lambda i: (i, 0)),
        ],
        out_specs=pl.BlockSpec((1, 1), lambda i: (0, 0)),
        out_shape=jax.ShapeDtypeStruct((1, 1), jnp.float32),
    )(tea, stu, tea_g, stu_g, perm, t, cstar, mt, ms)


def kernel(stu_dis, tea_dis):
    perm = jnp.asarray(_perm_table())
    tea_g, stu_g = _sc_gather()(tea_dis, stu_dis, perm)
    t, cstar, mt, ms = _tc_select(tea_dis, stu_dis)
    out = _tc_final(tea_dis, stu_dis, tea_g, stu_g, perm, t, cstar, mt, ms)
    return out[0, 0]


# speedup vs baseline: 38.6502x; 1.0017x over previous
import functools

import numpy as np
import jax
import jax.numpy as jnp
from jax import lax
from jax.experimental import pallas as pl
from jax.experimental.pallas import tpu as pltpu
from jax.experimental.pallas import tpu_sc as plsc

_B, _N = 128, 32768
_K = 1024
_M = 1024
_G = 2048
_R = 32
_NW = 32
_RPW = _B // _NW

_PERM = None


def _np_threefry_uniform(seed: int, shape) -> np.ndarray:
    def rotl(x, d):
        return ((x << np.uint32(d)) | (x >> np.uint32(32 - d))).astype(np.uint32)

    size = int(np.prod(shape))
    x0 = np.zeros(size, dtype=np.uint32)
    x1 = np.arange(size, dtype=np.uint32)
    ks0 = np.uint32(seed >> 32)
    ks1 = np.uint32(seed & 0xFFFFFFFF)
    ks2 = np.uint32(np.uint32(0x1BD11BDA) ^ ks0 ^ ks1)
    x0 = (x0 + ks0).astype(np.uint32)
    x1 = (x1 + ks1).astype(np.uint32)
    rotations = [(13, 15, 26, 6), (17, 29, 16, 24)]
    keys = [(ks1, ks2), (ks2, ks0), (ks0, ks1), (ks1, ks2), (ks2, ks0)]
    for i in range(5):
        for d in rotations[i % 2]:
            x0 = (x0 + x1).astype(np.uint32)
            x1 = rotl(x1, d)
            x1 = (x1 ^ x0).astype(np.uint32)
        x0 = (x0 + keys[i][0]).astype(np.uint32)
        x1 = (x1 + keys[i][1] + np.uint32(i + 1)).astype(np.uint32)
    bits = (x0 ^ x1).reshape(shape)
    fb = ((bits >> np.uint32(9)) | np.uint32(0x3F800000)).astype(np.uint32)
    return fb.view(np.float32) - np.float32(1.0)


def _perm_table() -> np.ndarray:
    global _PERM
    if _PERM is None:
        scores = _np_threefry_uniform(1234, (_B, _N - 1))
        order = np.argsort(-scores, axis=1, kind="stable")[:, :_G]
        _PERM = (order + 1).astype(np.int32)
    return _PERM


def _sc_gather_body(tea_hbm, stu_hbm, perm_hbm, tea_out, stu_out,
                    row_t, row_s, idx_v, gat_t, gat_s):
    c = lax.axis_index("c")
    s = lax.axis_index("s")
    wid = s * 2 + c
    for r in range(_RPW):
        row = wid * _RPW + r
        pltpu.sync_copy(perm_hbm.at[row], idx_v)
        pltpu.sync_copy(tea_hbm.at[row], row_t)
        pltpu.sync_copy(stu_hbm.at[row], row_s)

        def body(j, carry):
            sl = pl.ds(j * 16, 16)
            idx = idx_v[sl]
            gat_t[sl] = plsc.load_gather(row_t, [idx])
            gat_s[sl] = plsc.load_gather(row_s, [idx])
            return carry

        lax.fori_loop(0, _G // 16, body, 0)
        pltpu.sync_copy(gat_t, tea_out.at[row])
        pltpu.sync_copy(gat_s, stu_out.at[row])


_SC_GATHER = None


def _sc_gather():
    global _SC_GATHER
    if _SC_GATHER is None:
        _SC_GATHER = pl.kernel(
            _sc_gather_body,
            mesh=plsc.VectorSubcoreMesh(core_axis_name="c", subcore_axis_name="s"),
            compiler_params=pltpu.CompilerParams(needs_layout_passes=False),
            out_type=[
                jax.ShapeDtypeStruct((_B, _G), jnp.float32),
                jax.ShapeDtypeStruct((_B, _G), jnp.float32),
            ],
            scratch_types=[
                pltpu.VMEM((_N,), jnp.float32),
                pltpu.VMEM((_N,), jnp.float32),
                pltpu.VMEM((_G,), jnp.int32),
                pltpu.VMEM((_G,), jnp.float32),
                pltpu.VMEM((_G,), jnp.float32),
            ],
        )
    return _SC_GATHER


def _mono_key(x):
    bu = lax.bitcast_convert_type(x, jnp.uint32)
    neg = bu >= jnp.uint32(0x80000000)
    key = jnp.where(neg, ~bu, bu | jnp.uint32(0x80000000))
    return jnp.where(x == 0.0, jnp.uint32(0x80000000), key)


_CW1 = 512
_CW2 = 256


def _fold128(v, op):
    r = v[:, 0:128]
    for off in range(128, v.shape[1], 128):
        r = op(r, v[:, off:off + 128])
    return r


def _tc_select_body(tea_ref, stu_ref, t_ref, cstar_ref, mt_ref, ms_ref,
                    key_ref):
    nch1 = _N // _CW1
    iota1 = lax.broadcasted_iota(jnp.int32, (_R, _CW1), 1)
    add = lambda a, b: a + b

    mt_acc = jnp.full((_R, 128), -jnp.inf, jnp.float32)
    ms_acc = jnp.full((_R, 128), -jnp.inf, jnp.float32)
    for c in range(nch1):
        sl = pl.ds(c * _CW1, _CW1)
        x = tea_ref[:, sl]
        s = stu_ref[:, sl]
        k = _mono_key(x)
        if c == 0:
            k = jnp.where(iota1 == 0, jnp.uint32(0), k)
        key_ref[:, sl] = k
        mt_acc = jnp.maximum(mt_acc, _fold128(x, jnp.maximum))
        ms_acc = jnp.maximum(ms_acc, _fold128(s, jnp.maximum))
    mt = jnp.max(mt_acc, axis=1, keepdims=True)
    ms = jnp.max(ms_acc, axis=1, keepdims=True)

    def count(pred_fn):
        acc = jnp.zeros((_R, 128), jnp.float32)
        for c in range(nch1):
            kc = key_ref[:, pl.ds(c * _CW1, _CW1)]
            m = jnp.where(pred_fn(kc, c), 1.0, 0.0)
            acc = acc + _fold128(m, add)
        return jnp.sum(acc, axis=1, keepdims=True)

    half = _R // 2

    def count_h(r0, pred_fn):
        acc = jnp.zeros((half, 128), jnp.float32)
        for c in range(nch1):
            kc = key_ref[r0:r0 + half, pl.ds(c * _CW1, _CW1)]
            m = jnp.where(pred_fn(kc), 1.0, 0.0)
            acc = acc + _fold128(m, add)
        return jnp.sum(acc, axis=1, keepdims=True)

    def s1_body(i, carry):
        lo_a, hi_a, lo_b, hi_b = carry
        mid_a = lo_a + ((hi_a - lo_a + jnp.uint32(1)) >> 1)
        mid_b = lo_b + ((hi_b - lo_b + jnp.uint32(1)) >> 1)
        ge_a = count_h(0, lambda kc: kc >= mid_a) >= _K
        ge_b = count_h(half, lambda kc: kc >= mid_b) >= _K
        return (jnp.where(ge_a, mid_a, lo_a),
                jnp.where(ge_a, hi_a, mid_a - jnp.uint32(1)),
                jnp.where(ge_b, mid_b, lo_b),
                jnp.where(ge_b, hi_b, mid_b - jnp.uint32(1)))

    lo0 = jnp.zeros((half, 1), jnp.uint32)
    hi0 = jnp.full((half, 1), 0xFF800000, jnp.uint32)
    t_a, _, t_b, _ = lax.fori_loop(0, 32, s1_body, (lo0, hi0, lo0, hi0))
    t = jnp.concatenate([t_a, t_b], axis=0)

    accg = jnp.zeros((_R, 128), jnp.float32)
    acce = jnp.zeros((_R, 128), jnp.float32)
    for c in range(nch1):
        kc = key_ref[:, pl.ds(c * _CW1, _CW1)]
        accg = accg + _fold128(jnp.where(kc > t, 1.0, 0.0), add)
        acce = acce + _fold128(jnp.where(kc == t, 1.0, 0.0), add)
    g = jnp.sum(accg, axis=1, keepdims=True)
    e = jnp.sum(acce, axis=1, keepdims=True)
    need = jnp.float32(_K) - g

    def s2_search(_):
        def s2_body(i, carry):
            lo, hi = carry
            mid = (lo + hi) >> 1
            ge = count(
                lambda kc, c: (kc == t) & ((iota1 + c * _CW1) <= mid)
            ) >= need
            return jnp.where(ge, lo, mid + 1), jnp.where(ge, mid, hi)

        lo1 = jnp.ones((_R, 1), jnp.int32)
        hi1 = jnp.full((_R, 1), _N - 1, jnp.int32)
        cs, _ = lax.fori_loop(0, 15, s2_body, (lo1, hi1))
        return cs

    cstar = lax.cond(
        jnp.any(e != need),
        s2_search,
        lambda _: jnp.full((_R, 1), _N - 1, jnp.int32),
        operand=None,
    )

    t_ref[...] = t
    cstar_ref[...] = cstar
    mt_ref[...] = mt
    ms_ref[...] = ms


def _tc_final_body(tea_ref, stu_ref, teag_ref, stug_ref, perm_ref,
                   t_ref, cstar_ref, mt_ref, ms_ref, out_ref):
    pid = pl.program_id(0)
    add = lambda a, b: a + b
    t = t_ref[...]
    cstar = cstar_ref[...]
    mt = mt_ref[...]
    ms = ms_ref[...]

    teag = teag_ref[...]
    stug = stug_ref[...]
    perm = perm_ref[...]
    keyg = _mono_key(teag)
    unm = ~((keyg > t) | ((keyg == t) & (perm <= cstar)))
    posg = lax.broadcasted_iota(jnp.int32, (_R, _G), 1)

    def gcount(pred):
        return jnp.sum(jnp.where(pred, 1.0, 0.0), axis=1, keepdims=True)

    def s3_body(i, carry):
        lo, hi = carry
        mid = (lo + hi) >> 1
        ge = gcount(unm & (posg < mid)) >= _M
        return jnp.where(ge, lo, mid + 1), jnp.where(ge, mid, hi)

    lo2 = jnp.full((_R, 1), _M, jnp.int32)
    hi2 = jnp.full((_R, 1), _G, jnp.int32)
    pcut, _ = lax.fori_loop(0, 11, s3_body, (lo2, hi2))
    sel = unm & (posg < pcut)

    nch2 = _N // _CW2
    iota2 = lax.broadcasted_iota(jnp.int32, (_R, _CW2), 1)
    acc_et = jnp.zeros((_R, 128), jnp.float32)
    acc_es = jnp.zeros((_R, 128), jnp.float32)
    acc_ts = jnp.zeros((_R, 128), jnp.float32)
    for c in range(nch2):
        sl = pl.ds(c * _CW2, _CW2)
        x = tea_ref[:, sl]
        s = stu_ref[:, sl]
        kc = _mono_key(x)
        m = (kc > t) | ((kc == t) & ((iota2 + c * _CW2) <= cstar))
        if c == 0:
            m = m | (iota2 == 0)
        w = jnp.where(m, jnp.exp(x - mt), 0.0)
        acc_et = acc_et + _fold128(w, add)
        acc_ts = acc_ts + _fold128(w * (x - s), add)
        acc_es = acc_es + _fold128(jnp.where(m, jnp.exp(s - ms), 0.0), add)
    e_t = jnp.sum(acc_et, axis=1, keepdims=True)
    e_s = jnp.sum(acc_es, axis=1, keepdims=True)
    tsum = jnp.sum(acc_ts, axis=1, keepdims=True)

    wg = jnp.where(sel, jnp.exp(teag - mt), 0.0)
    e_t = e_t + jnp.sum(wg, axis=1, keepdims=True)
    tsum = tsum + jnp.sum(wg * (teag - stug), axis=1, keepdims=True)
    e_s = e_s + jnp.sum(jnp.where(sel, jnp.exp(stug - ms), 0.0),
                        axis=1, keepdims=True)

    kl = tsum / e_t + (ms + jnp.log(e_s)) - (mt + jnp.log(e_t))
    part = (jnp.sum(kl) * jnp.float32(1.0 / _B)).reshape(1, 1)

    @pl.when(pid == 0)
    def _():
        out_ref[...] = jnp.zeros((1, 1), jnp.float32)

    out_ref[...] += part


def _tc_select(tea, stu):
    return pl.pallas_call(
        _tc_select_body,
        grid=(_B // _R,),
        in_specs=[
            pl.BlockSpec((_R, _N), lambda i: (i, 0)),
            pl.BlockSpec((_R, _N), lambda i: (i, 0)),
        ],
        out_specs=[
            pl.BlockSpec((_R, 1), lambda i: (i, 0)),
            pl.BlockSpec((_R, 1), lambda i: (i, 0)),
            pl.BlockSpec((_R, 1), lambda i: (i, 0)),
            pl.BlockSpec((_R, 1), lambda i: (i, 0)),
        ],
        out_shape=[
            jax.ShapeDtypeStruct((_B, 1), jnp.uint32),
            jax.ShapeDtypeStruct((_B, 1), jnp.int32),
            jax.ShapeDtypeStruct((_B, 1), jnp.float32),
            jax.ShapeDtypeStruct((_B, 1), jnp.float32),
        ],
        scratch_shapes=[pltpu.VMEM((_R, _N), jnp.uint32)],
    )(tea, stu)


def _tc_final(tea, stu, tea_g, stu_g, perm, t, cstar, mt, ms):
    return pl.pallas_call(
        _tc_final_body,
        grid=(_B // _R,),
        in_specs=[
            pl.BlockSpec((_R, _N), lambda i: (i, 0)),
            pl.BlockSpec((_R, _N), lambda i: (i, 0)),
            pl.BlockSpec((_R, _G), lambda i: (i, 0)),
            pl.BlockSpec((_R, _G), lambda i: (i, 0)),
            pl.BlockSpec((_R, _G), lambda i: (i, 0)),
            pl.BlockSpec((_R, 1), lambda i: (i, 0)),
            pl.BlockSpec((_R, 1), lambda i: (i, 0)),
            pl.BlockSpec((_R, 1), lambda i: (i, 0)),
            pl.BlockSpec((_R, 1), lambda i: (i, 0)),
        ],
        out_specs=pl.BlockSpec((1, 1), lambda i: (0, 0)),
        out_shape=jax.ShapeDtypeStruct((1, 1), jnp.float32),
    )(tea, stu, tea_g, stu_g, perm, t, cstar, mt, ms)


def kernel(stu_dis, tea_dis):
    perm = jnp.asarray(_perm_table())
    tea_g, stu_g = _sc_gather()(tea_dis, stu_dis, perm)
    t, cstar, mt, ms = _tc_select(tea_dis, stu_dis)
    out = _tc_final(tea_dis, stu_dis, tea_g, stu_g, perm, t, cstar, mt, ms)
    return out[0, 0]
